# Initial kernel scaffold; baseline (speedup 1.0000x reference)
import jax
import jax.numpy as jnp
from jax.experimental import pallas as pl


def kernel(edge_index_writes, edge_index_cites, edge_index_affil, edge_index_domain, edge_label_index, emb_author, emb_paper, emb_inst, emb_domain, bbp_feat, Wself1, Wmsg1, Wself2, Wmsg2, Wc1, bc1, Wc2, bc2):
    raise NotImplementedError("write your pallas kernel here")


# speedup vs baseline: 3.2124x; 3.2124x over previous
import jax
import jax.numpy as jnp
from jax import lax
from jax.experimental import pallas as pl
from jax.experimental.pallas import tpu as pltpu
from jax.experimental.pallas import tpu_sc as plsc

D = 128
NC = 2
NS = 16
L = 16
KB = 512
F = 128


def _mesh():
    return plsc.VectorSubcoreMesh(core_axis_name="c", subcore_axis_name="s",
                                  num_cores=NC, num_subcores=NS)


_SC_PARAMS = pltpu.CompilerParams(needs_layout_passes=False)


def _make_segsum_kernel(cfgs, n_x, c_max):
    out_types = [jax.ShapeDtypeStruct((c["C"] * c["n_chunks"], D), jnp.float32)
                 for c in cfgs]
    scratch = [
        pltpu.VMEM_SHARED((c_max + L, D), jnp.float32),
        pltpu.VMEM((F,), jnp.int32),
        pltpu.VMEM((F,), jnp.int32),
        pltpu.VMEM((F, D), jnp.float32),
        pltpu.VMEM((KB,), jnp.int32),
        pltpu.VMEM((KB,), jnp.int32),
        pltpu.VMEM((L, D), jnp.float32),
        pltpu.VMEM((64, D), jnp.float32),
        pltpu.SemaphoreType.DMA,
    ]

    def body(*refs):
        xs = refs[:n_x]
        srcs = [refs[n_x + 2 * i] for i in range(len(cfgs))]
        dsts = [refs[n_x + 2 * i + 1] for i in range(len(cfgs))]
        p = n_x + 2 * len(cfgs)
        outs = refs[p:p + len(cfgs)]
        accum, srcbuf, dstbuf, rows, sedge, dedge, zeros, bounce, sem = \
            refs[p + len(cfgs):]

        cid = lax.axis_index("c")
        sid = lax.axis_index("s")
        z16 = jnp.zeros((L,), jnp.float32)
        iota16 = lax.iota(jnp.int32, L)

        def _zrow(i, _):
            for j in range(D // L):
                zeros[i, pl.ds(j * L, L)] = z16
            return 0
        lax.fori_loop(0, L, _zrow, 0)
        if any(c.get("ones") for c in cfgs):
            def _orow(i, _):
                for j in range(D // L):
                    rows[i, pl.ds(j * L, L)] = z16 + 1.0
                return 0
            lax.fori_loop(0, F, _orow, 0)

        for ci, cfg in enumerate(cfgs):
            C = cfg["C"]
            RT = C // NS
            Et = cfg["E_pad"] // NS
            x_hbm = xs[cfg["xi"]]
            ones_mode = cfg.get("ones", False)
            base = sid * RT

            for r in range(cfg["n_chunks"] // NC):
                lo = (NC * r + cid) * C

                def _zcp(j, _):
                    pltpu.sync_copy(zeros, accum.at[pl.ds(base + j * L, L)])
                    return 0
                lax.fori_loop(0, RT // L, _zcp, 0)
                @pl.when(sid == 0)
                def _():
                    pltpu.sync_copy(zeros, accum.at[pl.ds(C, L)])
                plsc.subcore_barrier()

                def _fire():
                    if not ones_mode:
                        pltpu.async_copy(x_hbm.at[srcbuf], rows, sem).wait()
                    pltpu.sync_copy(rows, accum.at[dstbuf], add=True)

                def _blk(b, n):
                    off = sid * Et + b * KB
                    if not ones_mode:
                        pltpu.sync_copy(srcs[ci].at[pl.ds(off, KB)], sedge)
                    pltpu.sync_copy(dsts[ci].at[pl.ds(off, KB)], dedge)

                    def _step(i, n):
                        d16 = dedge[pl.ds(i * L, L)]
                        dl = d16 - lo
                        m = (dl >= 0) & (dl < C)
                        mi = m.astype(jnp.int32)
                        csum = plsc.cumsum(mi)
                        pos = n + csum - mi
                        m_lo = m & (pos < F)
                        m_hi = m & (pos >= F)
                        if not ones_mode:
                            s16 = sedge[pl.ds(i * L, L)]
                            plsc.store_scatter(srcbuf, [pos], s16, mask=m_lo)
                        plsc.store_scatter(dstbuf, [pos], dl, mask=m_lo)
                        n2 = n + jnp.sum(mi)
                        fired = n2 >= F
                        @pl.when(fired)
                        def _():
                            _fire()
                        if not ones_mode:
                            s16b = sedge[pl.ds(i * L, L)]
                            plsc.store_scatter(srcbuf, [pos - F], s16b,
                                               mask=m_hi)
                        plsc.store_scatter(dstbuf, [pos - F], dl, mask=m_hi)
                        return jnp.where(fired, n2 - F, n2)

                    return lax.fori_loop(0, KB // L, _step, n)

                n = lax.fori_loop(0, Et // KB, _blk, jnp.int32(0))

                def _pad(j, _):
                    posj = iota16 + j * L
                    mt = posj >= n
                    if not ones_mode:
                        plsc.store_scatter(srcbuf, [posj], iota16, mask=mt)
                    plsc.store_scatter(dstbuf, [posj], iota16 * 0 + C, mask=mt)
                    return 0
                lax.fori_loop(0, F // L, _pad, 0)
                _fire()
                plsc.subcore_barrier()

                def _wcp(j, _):
                    pltpu.sync_copy(accum.at[pl.ds(base + j * 64, 64)], bounce)
                    pltpu.sync_copy(bounce,
                                    outs[ci].at[pl.ds(lo + base + j * 64, 64)])
                    return 0
                lax.fori_loop(0, RT // 64, _wcp, 0)
                tl = RT % 64
                if tl:
                    pltpu.sync_copy(accum.at[pl.ds(base + RT - tl, tl)],
                                    bounce.at[pl.ds(0, tl)])
                    pltpu.sync_copy(bounce.at[pl.ds(0, tl)],
                                    outs[ci].at[pl.ds(lo + base + RT - tl, tl)])
                plsc.subcore_barrier()

    return pl.kernel(body, out_type=out_types, mesh=_mesh(),
                     scratch_types=scratch, compiler_params=_SC_PARAMS)


def _gather3(xa2, xp2, bbp, eli0, eli1):
    B = eli0.shape[0]
    bw = B // (NC * NS)

    def body(xa_hbm, xp_hbm, bb_hbm, i0_hbm, i1_hbm, ga, gp, gb,
             idx_v, rows_v, sem):
        wid = lax.axis_index("s") * NC + lax.axis_index("c")
        base = wid * bw
        pltpu.sync_copy(i0_hbm.at[pl.ds(base, bw)], idx_v)
        pltpu.async_copy(xa_hbm.at[idx_v], rows_v, sem).wait()
        pltpu.sync_copy(rows_v, ga.at[pl.ds(base, bw)])
        pltpu.sync_copy(i1_hbm.at[pl.ds(base, bw)], idx_v)
        pltpu.async_copy(xp_hbm.at[idx_v], rows_v, sem).wait()
        pltpu.sync_copy(rows_v, gp.at[pl.ds(base, bw)])
        pltpu.async_copy(bb_hbm.at[idx_v], rows_v, sem).wait()
        pltpu.sync_copy(rows_v, gb.at[pl.ds(base, bw)])

    out = [jax.ShapeDtypeStruct((B, D), jnp.float32)] * 3
    scratch = [pltpu.VMEM((bw,), jnp.int32),
               pltpu.VMEM((bw, D), jnp.float32),
               pltpu.SemaphoreType.DMA]
    return pl.kernel(body, out_type=out, mesh=_mesh(),
                     scratch_types=scratch,
                     compiler_params=_SC_PARAMS)(xa2, xp2, bbp, eli0, eli1)


R = 400


def _inv(cref):
    return 1.0 / jnp.maximum(cref[...][:, 0:1], 1.0)


def _combine_ap(nb_small, relu, xa, xp, ssw_a, cw_a, ssa_a, ca_a,
                ssw_p, cw_p, ssc_p, cc_p, ssd_p, cd_p,
                wsa, wsp, wm_wa, wm_aff, wm_wp, wm_c, wm_d):
    n = xa.shape[0]
    nb = n // R

    def body(xa_r, xp_r, sswa_r, cwa_r, ssaa_r, caa_r,
             sswp_r, cwp_r, sscp_r, ccp_r, ssdp_r, cdp_r,
             wsa_r, wsp_r, wmwa_r, wmaff_r, wmwp_r, wmc_r, wmd_r,
             oa_r, op_r):
        i = pl.program_id(0)
        f32 = jnp.float32
        acc_a = jnp.dot(xa_r[...], wsa_r[...], preferred_element_type=f32)
        acc_a += jnp.dot(sswa_r[...] * _inv(cwa_r), wmwa_r[...],
                         preferred_element_type=f32)
        acc_p = jnp.dot(xp_r[...], wsp_r[...], preferred_element_type=f32)
        acc_p += jnp.dot(sswp_r[...] * _inv(cwp_r), wmwp_r[...],
                         preferred_element_type=f32)
        acc_p += jnp.dot(sscp_r[...] * _inv(ccp_r), wmc_r[...],
                         preferred_element_type=f32)

        def _fin(a, p):
            if relu:
                a, p = jnp.maximum(a, 0.0), jnp.maximum(p, 0.0)
            oa_r[...], op_r[...] = a, p

        @pl.when(i < nb_small)
        def _():
            a2 = acc_a + jnp.dot(ssaa_r[...] * _inv(caa_r), wmaff_r[...],
                                 preferred_element_type=f32)
            p2 = acc_p + jnp.dot(ssdp_r[...] * _inv(cdp_r), wmd_r[...],
                                 preferred_element_type=f32)
            _fin(a2, p2)

        @pl.when(i >= nb_small)
        def _():
            _fin(acc_a, acc_p)

    big = pl.BlockSpec((R, D), lambda i: (i, 0))
    cnt = big
    small = pl.BlockSpec((R, D), lambda i: (jnp.minimum(i, nb_small - 1), 0))
    csmall = small
    w = pl.BlockSpec((D, D), lambda i: (0, 0))
    return pl.pallas_call(
        body,
        grid=(nb,),
        in_specs=[big, big, big, cnt, small, csmall,
                  big, cnt, big, cnt, small, csmall,
                  w, w, w, w, w, w, w],
        out_specs=[big, big],
        out_shape=[jax.ShapeDtypeStruct((n, D), jnp.float32)] * 2,
    )(xa, xp, ssw_a, cw_a, ssa_a, ca_a, ssw_p, cw_p, ssc_p, cc_p,
      ssd_p, cd_p, wsa, wsp, wm_wa, wm_aff, wm_wp, wm_c, wm_d)


def _self_relu2(xi, xd, wi, wd):
    n = xi.shape[0]
    nb = n // R

    def body(xi_r, xd_r, wi_r, wd_r, oi_r, od_r):
        f32 = jnp.float32
        oi_r[...] = jnp.maximum(
            jnp.dot(xi_r[...], wi_r[...], preferred_element_type=f32), 0.0)
        od_r[...] = jnp.maximum(
            jnp.dot(xd_r[...], wd_r[...], preferred_element_type=f32), 0.0)

    big = pl.BlockSpec((R, D), lambda i: (i, 0))
    w = pl.BlockSpec((D, D), lambda i: (0, 0))
    return pl.pallas_call(
        body, grid=(nb,), in_specs=[big, big, w, w], out_specs=[big, big],
        out_shape=[jax.ShapeDtypeStruct((n, D), jnp.float32)] * 2,
    )(xi, xd, wi, wd)


def _classifier(ga, gp, gb, Wc1, bc1, Wc2, bc2):
    B = ga.shape[0]
    RB = 512
    nb = B // RB

    def body(ga_r, gp_r, gb_r, w1_r, b1_r, w2_r, b2_r, o_r):
        f32 = jnp.float32
        h = jnp.dot(ga_r[...], w1_r[0:D, :], preferred_element_type=f32)
        h += jnp.dot(gp_r[...], w1_r[D:2 * D, :], preferred_element_type=f32)
        h += jnp.dot(gb_r[...], w1_r[2 * D:3 * D, :], preferred_element_type=f32)
        h = jnp.maximum(h + b1_r[...], 0.0)
        o_r[...] = jnp.sum(h * w2_r[...], axis=1, keepdims=True) + b2_r[...]

    big = pl.BlockSpec((RB, D), lambda i: (i, 0))
    return pl.pallas_call(
        body, grid=(nb,),
        in_specs=[big, big, big,
                  pl.BlockSpec((3 * D, D), lambda i: (0, 0)),
                  pl.BlockSpec((1, D), lambda i: (0, 0)),
                  pl.BlockSpec((1, D), lambda i: (0, 0)),
                  pl.BlockSpec((1, 1), lambda i: (0, 0))],
        out_specs=pl.BlockSpec((RB, 1), lambda i: (i, 0)),
        out_shape=jax.ShapeDtypeStruct((B, 1), jnp.float32),
    )(ga, gp, gb, Wc1, bc1, Wc2, bc2)


def _pad_edges(e, e_pad):
    src = e[0].astype(jnp.int32)
    dst = e[1].astype(jnp.int32)
    pad = e_pad - src.shape[0]
    src = jnp.concatenate([src, jnp.full((pad,), -1, jnp.int32)])
    dst = jnp.concatenate([dst, jnp.full((pad,), -1, jnp.int32)])
    return src, dst


def _epad(e):
    blk = NS * KB
    return ((e + blk - 1) // blk) * blk


def kernel(edge_index_writes, edge_index_cites, edge_index_affil,
           edge_index_domain, edge_label_index, emb_author, emb_paper,
           emb_inst, emb_domain, bbp_feat, Wself1, Wmsg1, Wself2, Wmsg2,
           Wc1, bc1, Wc2, bc2):
    E_w = edge_index_writes.shape[1]
    E_c = edge_index_cites.shape[1]
    E_a = edge_index_affil.shape[1]
    E_d = edge_index_domain.shape[1]
    ew_s, ew_d = _pad_edges(edge_index_writes, _epad(E_w))
    ec_s, ec_d = _pad_edges(edge_index_cites, _epad(E_c))
    ea_s, ea_d = _pad_edges(edge_index_affil, _epad(E_a))
    ed_s, ed_d = _pad_edges(edge_index_domain, _epad(E_d))

    C_BIG, NCH_BIG = 12544, 4
    C_SML, NCH_SML = 5120, 2
    cfgs = [
        dict(xi=0, E_pad=_epad(E_w), C=C_BIG, n_chunks=NCH_BIG),
        dict(xi=1, E_pad=_epad(E_w), C=C_BIG, n_chunks=NCH_BIG),
        dict(xi=1, E_pad=_epad(E_c), C=C_BIG, n_chunks=NCH_BIG),
        dict(xi=2, E_pad=_epad(E_a), C=C_SML, n_chunks=NCH_SML),
        dict(xi=3, E_pad=_epad(E_d), C=C_SML, n_chunks=NCH_SML),
    ]
    cfgs_cnt = [dict(c, ones=True) for c in cfgs]
    edge_args = (ew_s, ew_d, ew_d, ew_s, ec_s, ec_d, ea_s, ea_d, ed_s, ed_d)

    k1 = _make_segsum_kernel(cfgs_cnt + cfgs, 4, C_BIG)
    (c_wp, c_wa, c_cp, c_aa, c_dp,
     ss_wp, ss_wa, ss_cp, ss_aa, ss_dp) = k1(
        emb_author, emb_paper, emb_inst, emb_domain,
        *edge_args, *edge_args)

    xa1, xp1 = _combine_ap(
        25, True, emb_author, emb_paper, ss_wa, c_wa, ss_aa, c_aa,
        ss_wp, c_wp, ss_cp, c_cp, ss_dp, c_dp,
        Wself1[0], Wself1[1], Wmsg1[1], Wmsg1[3], Wmsg1[0], Wmsg1[2], Wmsg1[4])
    xi1, xd1 = _self_relu2(emb_inst, emb_domain, Wself1[2], Wself1[3])

    k2 = _make_segsum_kernel(cfgs, 4, C_BIG)
    ss2_wp, ss2_wa, ss2_cp, ss2_aa, ss2_dp = k2(
        xa1, xp1, xi1, xd1, *edge_args)

    xa2, xp2 = _combine_ap(
        25, False, xa1, xp1, ss2_wa, c_wa, ss2_aa, c_aa,
        ss2_wp, c_wp, ss2_cp, c_cp, ss2_dp, c_dp,
        Wself2[0], Wself2[1], Wmsg2[1], Wmsg2[3], Wmsg2[0], Wmsg2[2], Wmsg2[4])

    eli0 = edge_label_index[0].astype(jnp.int32)
    eli1 = edge_label_index[1].astype(jnp.int32)
    ga, gp, gb = _gather3(xa2, xp2, bbp_feat, eli0, eli1)

    pred = _classifier(ga, gp, gb, Wc1, bc1.reshape(1, D),
                       Wc2.reshape(1, D), bc2.reshape(1, 1))
    return pred.reshape(-1)
